# baseline (speedup 1.0000x reference)
import dataclasses
import functools

import jax
import jax.numpy as jnp
from jax import lax
from jax.experimental import pallas as pl
from jax.experimental.pallas import tpu as pltpu
from jax.experimental.pallas import tpu_sc as plsc

NUM_ROWS = 1000000
FEAT = 64
BATCH = 4096
SEQ = 200
LANES = 16
NCHUNK = FEAT // LANES
NUM_CORES = 2
NUM_SUBCORES = 16
NUM_WORKERS = NUM_CORES * NUM_SUBCORES
SEQ_PER_W = BATCH // NUM_WORKERS
GCHUNK_A = 128
GCHUNK_B = SEQ - GCHUNK_A


def _rsqrt_newton(x):
    i = lax.bitcast_convert_type(x, jnp.int32)
    i = jnp.int32(0x5F3759DF) - lax.shift_right_logical(i, 1)
    y = lax.bitcast_convert_type(i, jnp.float32)
    for _ in range(3):
        y = y * (1.5 - (0.5 * x) * y * y)
    return y


def _make_sc_kernel():
    mesh = plsc.VectorSubcoreMesh(core_axis_name="c", subcore_axis_name="s")
    cp = pltpu.CompilerParams()
    if "needs_layout_passes" in pltpu.CompilerParams.__dataclass_fields__:
        cp = dataclasses.replace(cp, needs_layout_passes=False)

    @functools.partial(
        pl.kernel,
        mesh=mesh,
        compiler_params=cp,
        out_type=jax.ShapeDtypeStruct((BATCH * SEQ // 2, 2 * FEAT), jnp.float32),
        scratch_types=[
            pltpu.VMEM((SEQ, 2 * FEAT), jnp.float32),
            pltpu.VMEM((SEQ, 2 * FEAT), jnp.float32),
            pltpu.VMEM((SEQ, 2 * FEAT), jnp.float32),
            pltpu.VMEM((SEQ, 2 * FEAT), jnp.float32),
            pltpu.VMEM((SEQ,), jnp.int32),
            pltpu.VMEM((SEQ,), jnp.int32),
            pltpu.VMEM((SEQ,), jnp.int32),
            pltpu.VMEM((SEQ,), jnp.int32),
            pltpu.VMEM((SEQ,), jnp.int32),
            pltpu.VMEM((SEQ,), jnp.int32),
            pltpu.VMEM((SEQ,), jnp.int32),
            pltpu.VMEM((SEQ,), jnp.int32),
            pltpu.SemaphoreType.DMA,
            pltpu.SemaphoreType.DMA,
            pltpu.SemaphoreType.DMA,
            pltpu.SemaphoreType.DMA,
            pltpu.SemaphoreType.DMA,
            pltpu.SemaphoreType.DMA,
            pltpu.SemaphoreType.DMA,
            pltpu.SemaphoreType.DMA,
        ],
    )
    def sc_kernel(ih_hbm, par_hbm, table_hbm, out_hbm,
                  gb0, gb1, ob0, ob1, ih0, ih1, ih2, ih3, pr0, pr1, pr2, pr3,
                  g0, g1, w0, w1, i0, i1, i2, i3):
        gbufs = (gb0, gb1)
        obufs = (ob0, ob1)
        ihs = (ih0, ih1, ih2, ih3)
        prs = (pr0, pr1, pr2, pr3)
        gsems = (g0, g1)
        wsems = (w0, w1)
        isems = (i0, i1, i2, i3)

        wid = lax.axis_index("s") * NUM_CORES + lax.axis_index("c")
        base = wid * SEQ_PER_W

        inv_n = jnp.float32(1.0 / SEQ)
        inv_nm1 = jnp.float32(1.0 / (SEQ - 1))
        zero = jnp.zeros((LANES,), jnp.float32)
        iota = lax.iota(jnp.int32, LANES)
        colbase = [iota + LANES * c for c in range(NCHUNK)]
        zero_i = jnp.zeros((LANES,), jnp.int32)

        def istage(s, slot):
            off = (base + s) * SEQ
            pltpu.async_copy(ih_hbm.at[pl.ds(off, SEQ)], ihs[slot], isems[slot])
            pltpu.async_copy(par_hbm.at[pl.ds(off, SEQ)], prs[slot], isems[slot])

        def iwait(slot):
            pltpu.make_async_copy(ih_hbm.at[pl.ds(0, SEQ)], ihs[slot],
                                  isems[slot]).wait()
            pltpu.make_async_copy(par_hbm.at[pl.ds(0, SEQ)], prs[slot],
                                  isems[slot]).wait()

        def gstart(islot, gslot):
            pltpu.async_copy(
                table_hbm.at[ihs[islot].at[pl.ds(0, GCHUNK_A)]],
                gbufs[gslot].at[pl.ds(0, GCHUNK_A)],
                gsems[gslot],
            )
            pltpu.async_copy(
                table_hbm.at[ihs[islot].at[pl.ds(GCHUNK_A, GCHUNK_B)]],
                gbufs[gslot].at[pl.ds(GCHUNK_A, GCHUNK_B)],
                gsems[gslot],
            )

        def gwait(slot):
            pltpu.make_async_copy(out_hbm.at[pl.ds(0, SEQ)], gbufs[slot],
                                  gsems[slot]).wait()

        def wstart(row, slot):
            pltpu.async_copy(obufs[slot], out_hbm.at[pl.ds(row, SEQ)],
                             wsems[slot])

        def wwait(slot):
            pltpu.make_async_copy(obufs[slot], out_hbm.at[pl.ds(0, SEQ)],
                                  wsems[slot]).wait()

        def row_chunks(gb, pr, rr):
            rsplat = zero_i + rr
            offv = plsc.load_gather(pr, [rsplat])
            return [plsc.load_gather(gb, [rsplat, offv + colbase[c]])
                    for c in range(NCHUNK)]

        def compute(gslot, islot, oslot, ohalf):
            gb = gbufs[gslot]
            pr = prs[islot]
            ob = obufs[oslot]

            def p1(r2, carry):
                out = list(carry)
                for rr in (r2 * 2, r2 * 2 + 1):
                    vs = row_chunks(gb, pr, rr)
                    for c in range(NCHUNK):
                        out[2 * c] = out[2 * c] + vs[c]
                        out[2 * c + 1] = out[2 * c + 1] + vs[c] * vs[c]
                return tuple(out)

            acc = lax.fori_loop(0, SEQ // 2, p1, (zero,) * (2 * NCHUNK))

            scale = []
            shift = []
            for c in range(NCHUNK):
                sm = acc[2 * c]
                sq = acc[2 * c + 1]
                mean = sm * inv_n
                var = jnp.maximum((sq - sm * mean) * inv_nm1, 0.0)
                std = var * _rsqrt_newton(var)
                inv = 1.0 / (std + 1e-8)
                scale.append(inv)
                shift.append(-mean * inv)

            obase = ohalf * (SEQ // 2)

            def p2(r2, carry):
                for j, rr in enumerate((r2 * 2, r2 * 2 + 1)):
                    vs = row_chunks(gb, pr, rr)
                    for c in range(NCHUNK):
                        ob[obase + r2, pl.ds(j * FEAT + LANES * c, LANES)] = (
                            vs[c] * scale[c] + shift[c])
                return carry

            lax.fori_loop(0, SEQ // 2, p2, 0)

        for j in range(4):
            istage(j, j)
        iwait(0)
        gstart(0, 0)
        iwait(1)
        gstart(1, 1)

        obase_row = (base // 2) * SEQ

        @pl.loop(0, SEQ_PER_W, step=4)
        def _(s):
            for k in range(4):
                gslot = k % 2
                islot = k
                oslot = k // 2
                ohalf = k % 2
                gwait(gslot)
                if ohalf == 0:
                    @pl.when(s > 0)
                    def _():
                        wwait(oslot)

                compute(gslot, islot, oslot, ohalf)
                if ohalf == 1:
                    wstart(obase_row + s * (SEQ // 2) + (k - 1) * (SEQ // 2),
                           oslot)

                @pl.when(s + k + 4 < SEQ_PER_W)
                def _():
                    istage(s + k + 4, islot)

                @pl.when(s + k + 2 < SEQ_PER_W)
                def _():
                    iwait((k + 2) % 4)
                    gstart((k + 2) % 4, gslot)

        wwait(0)
        wwait(1)

    return sc_kernel


_SC_KERNEL = _make_sc_kernel()


@jax.jit
def kernel(index_tensor, features):
    idx = index_tensor.astype(jnp.int32).reshape(-1)
    idx_half = lax.shift_right_logical(idx, 1)
    par64 = lax.shift_left(jnp.bitwise_and(idx, 1), 6)
    table2 = features.reshape(NUM_ROWS // 2, 2 * FEAT)
    out = _SC_KERNEL(idx_half, par64, table2)
    return out.reshape(BATCH, SEQ, FEAT)


# speedup vs baseline: 1.3199x; 1.3199x over previous
import functools

import jax
import jax.numpy as jnp
from jax import lax
from jax.experimental import pallas as pl
from jax.experimental.pallas import tpu as pltpu
from jax.experimental.pallas import tpu_sc as plsc

NUM_ROWS = 1000000
FEAT = 64
BATCH = 4096
SEQ = 200
LANES = 16
NCHUNK = FEAT // LANES
NUM_CORES = 2
NUM_SUBCORES = 16
NUM_WORKERS = NUM_CORES * NUM_SUBCORES
SEQ_PER_W = BATCH // NUM_WORKERS
GCHUNK_A = 128
GCHUNK_B = SEQ - GCHUNK_A


def _rsqrt_newton(x):
    i = lax.bitcast_convert_type(x, jnp.int32)
    i = jnp.int32(0x5F3759DF) - lax.shift_right_logical(i, 1)
    y = lax.bitcast_convert_type(i, jnp.float32)
    for _ in range(3):
        y = y * (1.5 - (0.5 * x) * y * y)
    return y


def _make_sc_kernel():
    mesh = plsc.VectorSubcoreMesh(core_axis_name="c", subcore_axis_name="s")

    @functools.partial(
        pl.kernel,
        mesh=mesh,
        compiler_params=pltpu.CompilerParams(use_tc_tiling_on_sc=False),
        out_type=jax.ShapeDtypeStruct((BATCH, SEQ, FEAT), jnp.float32),
        scratch_types=[
            pltpu.VMEM((SEQ_PER_W, SEQ), jnp.int32),
            pltpu.VMEM((SEQ, FEAT), jnp.float32),
            pltpu.VMEM((SEQ, FEAT), jnp.float32),
            pltpu.VMEM((SEQ, FEAT), jnp.float32),
            pltpu.VMEM((SEQ, FEAT), jnp.float32),
            pltpu.SemaphoreType.DMA,
            pltpu.SemaphoreType.DMA,
            pltpu.SemaphoreType.DMA,
            pltpu.SemaphoreType.DMA,
            pltpu.SemaphoreType.DMA,
            pltpu.SemaphoreType.DMA,
            pltpu.SemaphoreType.DMA,
            pltpu.SemaphoreType.DMA,
        ],
    )
    def sc_kernel(idx_hbm, table_hbm, out_hbm, idx_v,
                  b0, b1, b2, b3, g0, g1, g2, g3, w0, w1, w2, w3):
        bufs = (b0, b1, b2, b3)
        gsems = (g0, g1, g2, g3)
        wsems = (w0, w1, w2, w3)

        wid = lax.axis_index("s") * NUM_CORES + lax.axis_index("c")
        base = wid * SEQ_PER_W

        pltpu.sync_copy(idx_hbm.at[pl.ds(base, SEQ_PER_W)], idx_v)

        inv_n = jnp.float32(1.0 / SEQ)
        inv_nm1 = jnp.float32(1.0 / (SEQ - 1))
        zero = jnp.zeros((LANES,), jnp.float32)

        def gstart(s, buf, sem):
            pltpu.async_copy(
                table_hbm.at[idx_v.at[s, pl.ds(0, GCHUNK_A)]],
                buf.at[pl.ds(0, GCHUNK_A)],
                sem,
            )
            pltpu.async_copy(
                table_hbm.at[idx_v.at[s, pl.ds(GCHUNK_A, GCHUNK_B)]],
                buf.at[pl.ds(GCHUNK_A, GCHUNK_B)],
                sem,
            )

        def gwait(buf, sem):
            pltpu.make_async_copy(out_hbm.at[0], buf, sem).wait()

        def wstart(s, buf, sem):
            pltpu.async_copy(buf, out_hbm.at[base + s], sem)

        def wwait(buf, sem):
            pltpu.make_async_copy(buf, out_hbm.at[0], sem).wait()

        def compute(buf):
            def p1(r2, carry):
                r = r2 * 2
                out = list(carry)
                for rr in (r, r + 1):
                    for c in range(NCHUNK):
                        v = buf[rr, pl.ds(LANES * c, LANES)]
                        out[2 * c] = out[2 * c] + v
                        out[2 * c + 1] = out[2 * c + 1] + v * v
                return tuple(out)

            acc = lax.fori_loop(0, SEQ // 2, p1, (zero,) * (2 * NCHUNK))

            scale = []
            shift = []
            for c in range(NCHUNK):
                sm = acc[2 * c]
                sq = acc[2 * c + 1]
                mean = sm * inv_n
                var = jnp.maximum((sq - sm * mean) * inv_nm1, 0.0)
                std = var * _rsqrt_newton(var)
                inv = 1.0 / (std + 1e-8)
                scale.append(inv)
                shift.append(-mean * inv)

            def p2(r2, carry):
                r = r2 * 2
                for rr in (r, r + 1):
                    for c in range(NCHUNK):
                        v = buf[rr, pl.ds(LANES * c, LANES)]
                        buf[rr, pl.ds(LANES * c, LANES)] = v * scale[c] + shift[c]
                return carry

            lax.fori_loop(0, SEQ // 2, p2, 0)

        gstart(0, bufs[0], gsems[0])
        gstart(1, bufs[1], gsems[1])
        gstart(2, bufs[2], gsems[2])

        @pl.loop(0, SEQ_PER_W, step=4)
        def _(s):
            for b in range(4):
                pb = (b - 1) % 4
                t = s + b + 3

                @pl.when(t < SEQ_PER_W)
                def _():
                    if b == 0:
                        @pl.when(s > 0)
                        def _():
                            wwait(bufs[pb], wsems[pb])
                    else:
                        wwait(bufs[pb], wsems[pb])
                    gstart(t, bufs[pb], gsems[pb])

                gwait(bufs[b], gsems[b])
                compute(bufs[b])
                wstart(s + b, bufs[b], wsems[b])

        for b in range(4):
            wwait(bufs[b], wsems[b])

    return sc_kernel


_SC_KERNEL = _make_sc_kernel()


@jax.jit
def kernel(index_tensor, features):
    idx = index_tensor.astype(jnp.int32)
    return _SC_KERNEL(idx, features)
